# Initial kernel scaffold; baseline (speedup 1.0000x reference)
import jax
import jax.numpy as jnp
from jax.experimental import pallas as pl


def kernel(x, edge_index, pos_edge_index, neg_edge_index, W1, b1, W2, b2, W_ep, b_ep):
    raise NotImplementedError("write your pallas kernel here")


# speedup vs baseline: 21.1376x; 21.1376x over previous
import functools

import jax
import jax.numpy as jnp
from jax import lax
from jax.experimental import pallas as pl
from jax.experimental.pallas import tpu as pltpu
from jax.experimental.pallas import tpu_sc as plsc

N = 10000
E = 320000
D_IN = 128
D_HID = 16
D_OUT = 32

NC = 2
NS = 16
NW = NC * NS
EPW = E // NW
NPAD = 10240
RPT = NPAD // NS


def _mesh():
    return plsc.VectorSubcoreMesh(
        core_axis_name="c", subcore_axis_name="s", num_cores=NC, num_subcores=NS
    )


_SC_PARAMS = pltpu.CompilerParams(
    needs_layout_passes=False, use_tc_tiling_on_sc=False
)


def _wid():
    return lax.axis_index("s") * NC + lax.axis_index("c")


def _deg_count(dst):
    CH = 2000

    @functools.partial(
        pl.kernel,
        out_type=jax.ShapeDtypeStruct((NW, NPAD), jnp.float32),
        mesh=_mesh(),
        compiler_params=_SC_PARAMS,
        scratch_types=[
            pltpu.VMEM((CH,), jnp.int32),
            pltpu.VMEM((NPAD,), jnp.float32),
        ],
    )
    def k(dst_hbm, out_hbm, idx_v, cnt_v):
        wid = _wid()
        zeros16 = jnp.zeros((16,), jnp.float32)

        def zbody(i, carry):
            cnt_v[pl.ds(i * 16, 16)] = zeros16
            return carry

        lax.fori_loop(0, NPAD // 16, zbody, None, unroll=8)

        ones16 = jnp.ones((16,), jnp.float32)
        base = wid * EPW

        def chunk(ci, carry):
            pltpu.sync_copy(dst_hbm.at[pl.ds(base + ci * CH, CH)], idx_v)

            def inner(j, c2):
                idx = idx_v[pl.ds(j * 16, 16)]
                plsc.addupdate_scatter(cnt_v, [idx], ones16)
                return c2

            lax.fori_loop(0, CH // 16, inner, None, unroll=8)
            return carry

        lax.fori_loop(0, EPW // CH, chunk, None)
        pltpu.sync_copy(cnt_v, out_hbm.at[wid])

    return k(dst)


def _segment_rows(hs, src, dst, d):
    CH = 80
    NCHUNK = EPW // CH

    @functools.partial(
        pl.kernel,
        out_type=jax.ShapeDtypeStruct((NC, NPAD, d), jnp.float32),
        mesh=_mesh(),
        compiler_params=_SC_PARAMS,
        scratch_types=[
            pltpu.VMEM_SHARED((NPAD, d), jnp.float32),
            pltpu.VMEM((CH,), jnp.int32),
            pltpu.VMEM((CH,), jnp.int32),
            pltpu.VMEM((CH, d), jnp.float32),
            pltpu.VMEM((RPT, d), jnp.float32),
            pltpu.SemaphoreType.DMA,
        ],
    )
    def k(hs_hbm, src_hbm, dst_hbm, out_hbm, agg_sh, si_v, di_v, rows_v, buf_v, sem):
        c = lax.axis_index("c")
        s = lax.axis_index("s")
        wid = s * NC + c
        zeros16 = jnp.zeros((16,), jnp.float32)

        def zbody(i, carry):
            for t in range(d // 16):
                buf_v[i, pl.ds(t * 16, 16)] = zeros16
            return carry

        lax.fori_loop(0, RPT, zbody, None, unroll=4)
        pltpu.sync_copy(buf_v, agg_sh.at[pl.ds(s * RPT, RPT)])
        plsc.subcore_barrier()

        base = wid * EPW

        def chunk(ci, carry):
            off = base + ci * CH
            pltpu.sync_copy(src_hbm.at[pl.ds(off, CH)], si_v)
            pltpu.sync_copy(dst_hbm.at[pl.ds(off, CH)], di_v)
            pltpu.async_copy(hs_hbm.at[si_v], rows_v, sem).wait()
            pltpu.sync_copy(rows_v, agg_sh.at[di_v], add=True)
            return carry

        lax.fori_loop(0, NCHUNK, chunk, None)
        plsc.subcore_barrier()

        pltpu.sync_copy(agg_sh.at[pl.ds(s * RPT, RPT)], buf_v)
        pltpu.sync_copy(buf_v, out_hbm.at[c, pl.ds(s * RPT, RPT)])

    return k(hs, src, dst)


def _edge_logits(a, cvec, p0, p1, n0, n1):
    CH = 2000
    NCHUNK = EPW // CH

    @functools.partial(
        pl.kernel,
        out_type=jax.ShapeDtypeStruct((2 * E,), jnp.float32),
        mesh=_mesh(),
        compiler_params=_SC_PARAMS,
        scratch_types=[
            pltpu.VMEM((N,), jnp.float32),
            pltpu.VMEM((N,), jnp.float32),
            pltpu.VMEM((CH,), jnp.int32),
            pltpu.VMEM((CH,), jnp.int32),
            pltpu.VMEM((CH,), jnp.float32),
        ],
    )
    def k(a_hbm, c_hbm, p0_hbm, p1_hbm, n0_hbm, n1_hbm, out_hbm,
          a_v, c_v, i0_v, i1_v, o_v):
        wid = _wid()
        pltpu.sync_copy(a_hbm, a_v)
        pltpu.sync_copy(c_hbm, c_v)
        base = wid * EPW

        def do_side(src0_hbm, src1_hbm, out_off):
            def chunk(ci, carry):
                off = base + ci * CH
                pltpu.sync_copy(src0_hbm.at[pl.ds(off, CH)], i0_v)
                pltpu.sync_copy(src1_hbm.at[pl.ds(off, CH)], i1_v)

                def inner(j, c2):
                    sl = pl.ds(j * 16, 16)
                    va = plsc.load_gather(a_v, [i0_v[sl]])
                    vc = plsc.load_gather(c_v, [i1_v[sl]])
                    o_v[sl] = va + vc
                    return c2

                lax.fori_loop(0, CH // 16, inner, None, unroll=4)
                pltpu.sync_copy(o_v, out_hbm.at[pl.ds(out_off + off, CH)])
                return carry

            lax.fori_loop(0, NCHUNK, chunk, None)

        do_side(p0_hbm, p1_hbm, 0)
        do_side(n0_hbm, n1_hbm, E)

    return k(a, cvec, p0, p1, n0, n1)


def _mm_body(x_ref, w_ref, o_ref):
    o_ref[...] = jnp.dot(x_ref[...], w_ref[...], preferred_element_type=jnp.float32)


def _prep1_body(degp_ref, xw_ref, dinv_ref, hs1_ref):
    deg = jnp.sum(degp_ref[...], axis=0) + 1.0
    dinv = lax.rsqrt(deg)
    dinv_ref[...] = dinv[:, None]
    hs1_ref[...] = xw_ref[...] * dinv[:N, None]


def _prep2_body(aggp_ref, hs1_ref, dinv_ref, b1_ref, w2_ref, hs2_ref):
    dv = dinv_ref[...][:N]
    agg = aggp_ref[0, :N, :] + aggp_ref[1, :N, :] + hs1_ref[...]
    h1 = jnp.maximum(agg * dv + b1_ref[...], 0.0)
    hs2_ref[...] = jnp.dot(h1, w2_ref[...], preferred_element_type=jnp.float32) * dv


def _final_body(aggp_ref, hs2_ref, dinv_ref, b2_ref, wep_ref, bep_ref,
                z_ref, a_ref, c_ref):
    dv = dinv_ref[...][:N]
    z = (aggp_ref[0, :N, :] + aggp_ref[1, :N, :] + hs2_ref[...]) * dv + b2_ref[...]
    z_ref[...] = z
    wep = wep_ref[...]
    a_ref[...] = jnp.dot(z, wep[:D_OUT, :], preferred_element_type=jnp.float32) + bep_ref[...]
    c_ref[...] = jnp.dot(z, wep[D_OUT:, :], preferred_element_type=jnp.float32)


def _sds(shape):
    return jax.ShapeDtypeStruct(shape, jnp.float32)


def kernel(x, edge_index, pos_edge_index, neg_edge_index, W1, b1, W2, b2, W_ep, b_ep):
    src = edge_index[0]
    dst = edge_index[1]

    degp = _deg_count(dst)
    xw = pl.pallas_call(_mm_body, out_shape=_sds((N, D_HID)))(x, W1)

    dinv, hs1 = pl.pallas_call(
        _prep1_body, out_shape=(_sds((NPAD, 1)), _sds((N, D_HID)))
    )(degp, xw)

    agg1 = _segment_rows(hs1, src, dst, D_HID)

    hs2 = pl.pallas_call(_prep2_body, out_shape=_sds((N, D_OUT)))(
        agg1, hs1, dinv, b1.reshape(1, D_HID), W2
    )

    agg2 = _segment_rows(hs2, src, dst, D_OUT)

    z, a_col, c_col = pl.pallas_call(
        _final_body, out_shape=(_sds((N, D_OUT)), _sds((N, 1)), _sds((N, 1)))
    )(agg2, hs2, dinv, b2.reshape(1, D_OUT), W_ep, b_ep.reshape(1, 1))

    logits = _edge_logits(
        a_col.reshape(N), c_col.reshape(N),
        pos_edge_index[0], pos_edge_index[1],
        neg_edge_index[0], neg_edge_index[1],
    )
    return z, logits.reshape(2 * E, 1)


# speedup vs baseline: 53.1866x; 2.5162x over previous
import functools

import jax
import jax.numpy as jnp
from jax import lax
from jax.experimental import pallas as pl
from jax.experimental.pallas import tpu as pltpu
from jax.experimental.pallas import tpu_sc as plsc

N = 10000
E = 320000
D_IN = 128
D_HID = 16
D_OUT = 32

NC = 2
NS = 16
NW = NC * NS
EPW = E // NW
NPAD = 10240
RPT = NPAD // NS


def _mesh():
    return plsc.VectorSubcoreMesh(
        core_axis_name="c", subcore_axis_name="s", num_cores=NC, num_subcores=NS
    )


_SC_PARAMS = pltpu.CompilerParams(
    needs_layout_passes=False, use_tc_tiling_on_sc=False
)


def _wid():
    return lax.axis_index("s") * NC + lax.axis_index("c")


def _deg_count(dst):
    CH = 2000

    @functools.partial(
        pl.kernel,
        out_type=jax.ShapeDtypeStruct((NW, NPAD), jnp.float32),
        mesh=_mesh(),
        compiler_params=_SC_PARAMS,
        scratch_types=[
            pltpu.VMEM((CH,), jnp.int32),
            pltpu.VMEM((NPAD,), jnp.float32),
        ],
    )
    def k(dst_hbm, out_hbm, idx_v, cnt_v):
        wid = _wid()
        zeros16 = jnp.zeros((16,), jnp.float32)

        def zbody(i, carry):
            cnt_v[pl.ds(i * 16, 16)] = zeros16
            return carry

        lax.fori_loop(0, NPAD // 16, zbody, None, unroll=8)

        ones16 = jnp.ones((16,), jnp.float32)
        base = wid * EPW

        def chunk(ci, carry):
            pltpu.sync_copy(dst_hbm.at[pl.ds(base + ci * CH, CH)], idx_v)

            def inner(j, c2):
                idx = idx_v[pl.ds(j * 16, 16)]
                plsc.addupdate_scatter(cnt_v, [idx], ones16)
                return c2

            lax.fori_loop(0, CH // 16, inner, None, unroll=8)
            return carry

        lax.fori_loop(0, EPW // CH, chunk, None)
        pltpu.sync_copy(cnt_v, out_hbm.at[wid])

    return k(dst)


def _segment_rows(hs, src, dst, d):
    CH = 2000
    NCHUNK = EPW // CH

    @functools.partial(
        pl.kernel,
        out_type=jax.ShapeDtypeStruct((NC, NPAD, d), jnp.float32),
        mesh=_mesh(),
        compiler_params=_SC_PARAMS,
        scratch_types=[
            pltpu.VMEM_SHARED((NPAD, d), jnp.float32),
            pltpu.VMEM((CH,), jnp.int32),
            pltpu.VMEM((CH,), jnp.int32),
            pltpu.VMEM((CH, d), jnp.float32),
            pltpu.VMEM((RPT, d), jnp.float32),
            pltpu.SemaphoreType.DMA,
        ],
    )
    def k(hs_hbm, src_hbm, dst_hbm, out_hbm, agg_sh, si_v, di_v, rows_v, buf_v, sem):
        c = lax.axis_index("c")
        s = lax.axis_index("s")
        wid = s * NC + c
        zeros16 = jnp.zeros((16,), jnp.float32)

        def zbody(i, carry):
            for t in range(d // 16):
                buf_v[i, pl.ds(t * 16, 16)] = zeros16
            return carry

        lax.fori_loop(0, RPT, zbody, None, unroll=4)
        pltpu.sync_copy(buf_v, agg_sh.at[pl.ds(s * RPT, RPT)])
        plsc.subcore_barrier()

        base = wid * EPW

        def chunk(ci, carry):
            off = base + ci * CH
            pltpu.sync_copy(src_hbm.at[pl.ds(off, CH)], si_v)
            pltpu.sync_copy(dst_hbm.at[pl.ds(off, CH)], di_v)
            pltpu.async_copy(hs_hbm.at[si_v], rows_v, sem).wait()
            pltpu.sync_copy(rows_v, agg_sh.at[di_v], add=True)
            return carry

        lax.fori_loop(0, NCHUNK, chunk, None)
        plsc.subcore_barrier()

        pltpu.sync_copy(agg_sh.at[pl.ds(s * RPT, RPT)], buf_v)
        pltpu.sync_copy(buf_v, out_hbm.at[c, pl.ds(s * RPT, RPT)])

    return k(hs, src, dst)


def _edge_logits(a, cvec, p0, p1, n0, n1):
    CH = 2000
    NCHUNK = EPW // CH

    @functools.partial(
        pl.kernel,
        out_type=jax.ShapeDtypeStruct((2 * E,), jnp.float32),
        mesh=_mesh(),
        compiler_params=_SC_PARAMS,
        scratch_types=[
            pltpu.VMEM((N,), jnp.float32),
            pltpu.VMEM((N,), jnp.float32),
            pltpu.VMEM((CH,), jnp.int32),
            pltpu.VMEM((CH,), jnp.int32),
            pltpu.VMEM((CH,), jnp.float32),
        ],
    )
    def k(a_hbm, c_hbm, p0_hbm, p1_hbm, n0_hbm, n1_hbm, out_hbm,
          a_v, c_v, i0_v, i1_v, o_v):
        wid = _wid()
        pltpu.sync_copy(a_hbm, a_v)
        pltpu.sync_copy(c_hbm, c_v)
        base = wid * EPW

        def do_side(src0_hbm, src1_hbm, out_off):
            def chunk(ci, carry):
                off = base + ci * CH
                pltpu.sync_copy(src0_hbm.at[pl.ds(off, CH)], i0_v)
                pltpu.sync_copy(src1_hbm.at[pl.ds(off, CH)], i1_v)

                def inner(j, c2):
                    sl = pl.ds(j * 16, 16)
                    va = plsc.load_gather(a_v, [i0_v[sl]])
                    vc = plsc.load_gather(c_v, [i1_v[sl]])
                    o_v[sl] = va + vc
                    return c2

                lax.fori_loop(0, CH // 16, inner, None, unroll=4)
                pltpu.sync_copy(o_v, out_hbm.at[pl.ds(out_off + off, CH)])
                return carry

            lax.fori_loop(0, NCHUNK, chunk, None)

        do_side(p0_hbm, p1_hbm, 0)
        do_side(n0_hbm, n1_hbm, E)

    return k(a, cvec, p0, p1, n0, n1)


def _mm_body(x_ref, w_ref, o_ref):
    o_ref[...] = jnp.dot(x_ref[...], w_ref[...], preferred_element_type=jnp.float32)


def _prep1_body(degp_ref, xw_ref, dinv_ref, hs1_ref):
    deg = jnp.sum(degp_ref[...], axis=0) + 1.0
    dinv = lax.rsqrt(deg)
    dinv_ref[...] = dinv[:, None]
    hs1_ref[...] = xw_ref[...] * dinv[:N, None]


def _prep2_body(aggp_ref, hs1_ref, dinv_ref, b1_ref, w2_ref, hs2_ref):
    dv = dinv_ref[...][:N]
    agg = aggp_ref[0, :N, :] + aggp_ref[1, :N, :] + hs1_ref[...]
    h1 = jnp.maximum(agg * dv + b1_ref[...], 0.0)
    hs2_ref[...] = jnp.dot(h1, w2_ref[...], preferred_element_type=jnp.float32) * dv


def _final_body(aggp_ref, hs2_ref, dinv_ref, b2_ref, wep_ref, bep_ref,
                z_ref, a_ref, c_ref):
    dv = dinv_ref[...][:N]
    z = (aggp_ref[0, :N, :] + aggp_ref[1, :N, :] + hs2_ref[...]) * dv + b2_ref[...]
    z_ref[...] = z
    wep = wep_ref[...]
    a_ref[...] = jnp.dot(z, wep[:D_OUT, :], preferred_element_type=jnp.float32) + bep_ref[...]
    c_ref[...] = jnp.dot(z, wep[D_OUT:, :], preferred_element_type=jnp.float32)


def _sds(shape):
    return jax.ShapeDtypeStruct(shape, jnp.float32)


def kernel(x, edge_index, pos_edge_index, neg_edge_index, W1, b1, W2, b2, W_ep, b_ep):
    src = edge_index[0]
    dst = edge_index[1]

    degp = _deg_count(dst)
    xw = pl.pallas_call(_mm_body, out_shape=_sds((N, D_HID)))(x, W1)

    dinv, hs1 = pl.pallas_call(
        _prep1_body, out_shape=(_sds((NPAD, 1)), _sds((N, D_HID)))
    )(degp, xw)

    agg1 = _segment_rows(hs1, src, dst, D_HID)

    hs2 = pl.pallas_call(_prep2_body, out_shape=_sds((N, D_OUT)))(
        agg1, hs1, dinv, b1.reshape(1, D_HID), W2
    )

    agg2 = _segment_rows(hs2, src, dst, D_OUT)

    z, a_col, c_col = pl.pallas_call(
        _final_body, out_shape=(_sds((N, D_OUT)), _sds((N, 1)), _sds((N, 1)))
    )(agg2, hs2, dinv, b2.reshape(1, D_OUT), W_ep, b_ep.reshape(1, 1))

    logits = _edge_logits(
        a_col.reshape(N), c_col.reshape(N),
        pos_edge_index[0], pos_edge_index[1],
        neg_edge_index[0], neg_edge_index[1],
    )
    return z, logits.reshape(2 * E, 1)
